# baseline (speedup 1.0000x reference)
import functools

import jax
import jax.numpy as jnp
from jax.experimental import pallas as pl

NB = 255
LOW = -20.0
STEP = 40.0 / 254.0
INVSTEP = 254.0 / 40.0


def _twohot_tile(vt_ref, bins_ref, out_ref):
    vt = vt_ref[...]
    b = bins_ref[0, :]
    vc = jnp.clip(vt, b[0], b[NB - 1])
    t = jnp.sign(vc) * jnp.log1p(jnp.abs(vc))
    ti = (t - LOW) * INVSTEP
    li = jnp.clip(jnp.floor(ti).astype(jnp.int32), 0, NB - 2)
    lx = LOW + li.astype(jnp.float32) * STEP
    rx = lx + STEP
    lv = jnp.sign(lx) * (jnp.exp(jnp.abs(lx)) - 1.0)
    rv = jnp.sign(rx) * (jnp.exp(jnp.abs(rx)) - 1.0)
    rw = (vc - lv) / (rv - lv + 1e-08)
    lw = 1.0 - rw
    liT = li.T
    lwT = lw.T
    rwT = rw.T
    jj = jax.lax.broadcasted_iota(jnp.int32, (1, 1, NB), 2)
    u = jj - liT[:, :, None]
    zero = jnp.zeros((), jnp.float32)
    out_ref[...] = jnp.where(u == 0, lwT[:, :, None],
                             jnp.where(u == 1, rwT[:, :, None], zero))


@functools.partial(jax.jit, static_argnames=("bcols",))
def _twohot(values, bins, bcols=64):
    nrows, ncols = values.shape
    gj = ncols // bcols
    vt = values.T
    bins2 = bins.reshape(1, NB)
    out = pl.pallas_call(
        _twohot_tile,
        grid=(gj,),
        in_specs=[
            pl.BlockSpec((bcols, nrows), lambda j: (j, 0)),
            pl.BlockSpec((1, NB), lambda j: (0, 0)),
        ],
        out_specs=pl.BlockSpec((nrows, bcols, NB), lambda j: (0, j, 0)),
        out_shape=jax.ShapeDtypeStruct((nrows, ncols, NB), jnp.float32),
    )(vt, bins2)
    return out


def kernel(values, bins):
    return _twohot(values, bins)
